# Initial kernel scaffold; baseline (speedup 1.0000x reference)
import jax
import jax.numpy as jnp
from jax.experimental import pallas as pl


def kernel(center, pos, neg, center_emb, context_emb):
    raise NotImplementedError("write your pallas kernel here")


# speedup vs baseline: 5.1404x; 5.1404x over previous
import functools

import jax
import jax.numpy as jnp
from jax import lax
from jax.experimental import pallas as pl
from jax.experimental.pallas import tpu as pltpu
from jax.experimental.pallas import tpu_sc as plsc

NC = 2
NS = 16
NW = NC * NS
L = 16

CB = 32
IDX_CH = 128


def _make_kernel(B, K, D, OUTW):
  b_per_w = B // NW
  n_chunks = b_per_w // CB
  nk = CB * K
  n_idx_ch = nk // IDX_CH
  nd = D // L

  mesh = plsc.VectorSubcoreMesh(core_axis_name="c", subcore_axis_name="s")

  def body(center_hbm, pos_hbm, negf_hbm, cemb_hbm, xemb_hbm, out_hbm,
           cidx, pidx, nidx, vrows, prows, nrows, outbuf, sem):
    wid = lax.axis_index("s") * NC + lax.axis_index("c")
    base = wid * b_per_w
    lane = lax.iota(jnp.int32, L)
    perms = [(lane ^ s).reshape(L, 1) for s in (8, 4, 2, 1)]
    dnums = lax.GatherDimensionNumbers(
        offset_dims=(), collapsed_slice_dims=(0,), start_index_map=(0,))

    def lanesum(x):
      for idx in perms:
        x = x + lax.gather(x, idx, dnums, (1,),
                           mode=lax.GatherScatterMode.PROMISE_IN_BOUNDS)
      return x

    def chunk_body(c, carry):
      row0 = base + c * CB
      pltpu.sync_copy(center_hbm.at[pl.ds(row0, CB)], cidx)
      pltpu.sync_copy(pos_hbm.at[pl.ds(row0, CB)], pidx)
      pltpu.sync_copy(negf_hbm.at[pl.ds(row0 * K, nk)], nidx)
      cps = []
      cps.append(pltpu.async_copy(cemb_hbm.at[cidx], vrows, sem))
      cps.append(pltpu.async_copy(xemb_hbm.at[pidx], prows, sem))
      for j in range(n_idx_ch):
        cps.append(pltpu.async_copy(
            xemb_hbm.at[nidx.at[pl.ds(j * IDX_CH, IDX_CH)]],
            nrows.at[pl.ds(j * IDX_CH, IDX_CH)], sem))
      for cp in cps:
        cp.wait()

      def elem_body(b, carry2):
        v = [vrows[b, pl.ds(j * L, L)] for j in range(nd)]
        acc1 = jnp.zeros((L,), jnp.float32)
        acc2 = jnp.zeros((L,), jnp.float32)
        nbase = b * K
        for k in range(K):
          u = [nrows[nbase + k, pl.ds(j * L, L)] for j in range(nd)]
          p = v[0] * u[0]
          for j in range(1, nd):
            p = p + v[j] * u[j]
          s = lanesum(p)
          if k < L:
            acc1 = jnp.where(lane == k, s, acc1)
          else:
            acc2 = jnp.where(lane == (k - L), s, acc2)
        u = [prows[b, pl.ds(j * L, L)] for j in range(nd)]
        p = v[0] * u[0]
        for j in range(1, nd):
          p = p + v[j] * u[j]
        s = lanesum(p)
        acc2 = jnp.where(lane == (K - L), s, acc2)
        outbuf[b, pl.ds(0, L)] = acc1
        outbuf[b, pl.ds(L, L)] = acc2
        return carry2

      lax.fori_loop(0, CB, elem_body, 0)
      pltpu.sync_copy(outbuf, out_hbm.at[pl.ds(row0, CB)])
      return carry

    lax.fori_loop(0, n_chunks, chunk_body, 0)

  return pl.kernel(
      body,
      out_type=jax.ShapeDtypeStruct((B, OUTW), jnp.float32),
      mesh=mesh,
      scratch_types=[
          pltpu.VMEM((CB,), jnp.int32),
          pltpu.VMEM((CB,), jnp.int32),
          pltpu.VMEM((nk,), jnp.int32),
          pltpu.VMEM((CB, D), jnp.float32),
          pltpu.VMEM((CB, D), jnp.float32),
          pltpu.VMEM((nk, D), jnp.float32),
          pltpu.VMEM((CB, OUTW), jnp.float32),
          pltpu.SemaphoreType.DMA,
      ],
      compiler_params=pltpu.CompilerParams(use_tc_tiling_on_sc=False),
  )


def kernel(center, pos, neg, center_emb, context_emb):
  B = center.shape[0]
  K = neg.shape[1]
  D = center_emb.shape[1]
  OUTW = 32
  center = center.astype(jnp.int32)
  pos = pos.astype(jnp.int32)
  negf = neg.astype(jnp.int32).reshape(B * K)
  k = _make_kernel(B, K, D, OUTW)
  out = k(center, pos, negf, center_emb, context_emb)
  return out[:, K], out[:, :K]


# speedup vs baseline: 10.3233x; 2.0083x over previous
import functools

import jax
import jax.numpy as jnp
from jax import lax
from jax.experimental import pallas as pl
from jax.experimental.pallas import tpu as pltpu
from jax.experimental.pallas import tpu_sc as plsc

NC = 2
NS = 16
NW = NC * NS
L = 16

CB = 32
IDX_CH = 128
BW = 8192

_DNUMS = lax.GatherDimensionNumbers(
    offset_dims=(), collapsed_slice_dims=(0,), start_index_map=(0,))


def _gat(x, idx):
  return lax.gather(x, idx, _DNUMS, (1,),
                    mode=lax.GatherScatterMode.PROMISE_IN_BOUNDS)


def _make_relayout(V, D):
  TD = 2 * D
  n_blocks = (V + BW - 1) // BW
  H = n_blocks * (BW // 2)

  def tbody(a_ref, b_ref, pa_ref, pb_ref):
    r = lax.broadcasted_iota(jnp.int32, (TD, TD), 0)
    c = lax.broadcasted_iota(jnp.int32, (TD, TD), 1)
    ident = jnp.where(r == c, 1.0, 0.0).astype(jnp.float32)
    dn = (((1,), (1,)), ((), ()))

    def dott(x):
      return lax.dot_general(ident, x, dn, preferred_element_type=jnp.float32)

    ya = jnp.swapaxes(a_ref[...], 0, 1)
    xb = b_ref[...]
    for j in range(BW // (2 * TD)):
      pa_ref[j * TD:(j + 1) * TD, :] = jnp.concatenate(
          [ya[j * 2 * TD:j * 2 * TD + TD], ya[j * 2 * TD + TD:(j + 1) * 2 * TD]],
          axis=1)
      yt = dott(xb[:, j * 2 * TD:j * 2 * TD + TD])
      yb2 = dott(xb[:, j * 2 * TD + TD:(j + 1) * 2 * TD])
      pb_ref[j * TD:(j + 1) * TD, :] = jnp.concatenate([yt, yb2], axis=1)

  return pl.pallas_call(
      tbody,
      grid=(n_blocks,),
      in_specs=[
          pl.BlockSpec((D, BW), lambda g: (0, g)),
          pl.BlockSpec((D, BW), lambda g: (0, g)),
      ],
      out_specs=[
          pl.BlockSpec((BW // 2, TD), lambda g: (g, 0)),
          pl.BlockSpec((BW // 2, TD), lambda g: (g, 0)),
      ],
      out_shape=[
          jax.ShapeDtypeStruct((H, TD), jnp.float32),
          jax.ShapeDtypeStruct((H, TD), jnp.float32),
      ],
      compiler_params=pltpu.CompilerParams(needs_layout_passes=False),
  )


def _make_sc_kernel(B, K, D, OUTW, H):
  TD = 2 * D
  b_per_w = B // NW
  n_chunks = b_per_w // CB
  nk = CB * K
  n_idx_ch = nk // IDX_CH
  nd = D // L

  mesh = plsc.VectorSubcoreMesh(core_axis_name="c", subcore_axis_name="s")

  seg = [0]
  for t in range(n_idx_ch):
    seg.append(min(CB, (IDX_CH * (t + 1)) // K))

  def body(center_hbm, pos_hbm, negf_hbm, cemb_hbm, xemb_hbm, out_hbm,
           cidx, pidx, nidx, crow, prow, nrow, vrows, prows, nrows, outbuf,
           sem_v, sem_p, *sem_n):
    wid = lax.axis_index("s") * NC + lax.axis_index("c")
    base = wid * b_per_w
    lane = lax.iota(jnp.int32, L)
    zero_idx = jnp.zeros((L, 1), jnp.int32)
    perms = [(lane ^ s).reshape(L, 1) for s in (8, 4, 2, 1)]
    kidx = [jnp.full((L, 1), kk, jnp.int32) for kk in range(L)]

    def lanesum(x):
      for idx in perms:
        x = x + _gat(x, idx)
      return x

    def pairrow(x):
      return ((x >> 8) << 7) | (x & 127)

    def chunk_body(c, carry):
      row0 = base + c * CB
      pltpu.sync_copy(center_hbm.at[pl.ds(row0, CB)], cidx.at[pl.ds(0, CB)])
      pltpu.sync_copy(pos_hbm.at[pl.ds(row0, CB)], pidx.at[pl.ds(0, CB)])
      pltpu.sync_copy(negf_hbm.at[pl.ds(row0 * K, nk)], nidx.at[pl.ds(0, nk)])
      for j in range(CB // L):
        crow[pl.ds(j * L, L)] = pairrow(cidx[pl.ds(j * L, L)])
        prow[pl.ds(j * L, L)] = pairrow(pidx[pl.ds(j * L, L)])
      for j in range(nk // L):
        nrow[pl.ds(j * L, L)] = pairrow(nidx[pl.ds(j * L, L)])
      cp_v = pltpu.async_copy(cemb_hbm.at[crow], vrows, sem_v)
      cp_p = pltpu.async_copy(xemb_hbm.at[prow], prows, sem_p)
      cp_n = []
      for j in range(n_idx_ch):
        cp_n.append(pltpu.async_copy(
            xemb_hbm.at[nrow.at[pl.ds(j * IDX_CH, IDX_CH)]],
            nrows.at[pl.ds(j * IDX_CH, IDX_CH)], sem_n[j]))
      cp_v.wait()
      cp_p.wait()

      def halfsel(t, ref, r):
        return [ref[r, pl.ds(j * L, L)] +
                t * (ref[r, pl.ds(D + j * L, L)] - ref[r, pl.ds(j * L, L)])
                for j in range(nd)]

      def elem_body(b, carry2):
        cpar = _gat(((cidx[pl.ds(b, L)] >> 7) & 1).astype(jnp.float32),
                    zero_idx)
        v = halfsel(cpar, vrows, b)
        pv1 = ((nidx[pl.ds(b * K, L)] >> 7) & 1).astype(jnp.float32)
        pv2 = ((nidx[pl.ds(b * K + L, L)] >> 7) & 1).astype(jnp.float32)
        acc1 = jnp.zeros((L,), jnp.float32)
        acc2 = jnp.zeros((L,), jnp.float32)
        nbase = b * K
        for k in range(K):
          kk = k if k < L else k - L
          pk = _gat(pv1 if k < L else pv2, kidx[kk])
          u = halfsel(pk, nrows, nbase + k)
          p = v[0] * u[0]
          for j in range(1, nd):
            p = p + v[j] * u[j]
          s = lanesum(p)
          if k < L:
            acc1 = jnp.where(lane == kk, s, acc1)
          else:
            acc2 = jnp.where(lane == kk, s, acc2)
        ppar = _gat(((pidx[pl.ds(b, L)] >> 7) & 1).astype(jnp.float32),
                    zero_idx)
        u = halfsel(ppar, prows, b)
        p = v[0] * u[0]
        for j in range(1, nd):
          p = p + v[j] * u[j]
        s = lanesum(p)
        acc2 = jnp.where(lane == (K - L), s, acc2)
        outbuf[b, pl.ds(0, L)] = acc1
        outbuf[b, pl.ds(L, L)] = acc2
        return carry2

      for t in range(n_idx_ch):
        cp_n[t].wait()
        if seg[t + 1] > seg[t]:
          lax.fori_loop(seg[t], seg[t + 1], elem_body, 0)
      pltpu.sync_copy(outbuf, out_hbm.at[pl.ds(row0, CB)])
      return carry

    lax.fori_loop(0, n_chunks, chunk_body, 0)

  return pl.kernel(
      body,
      out_type=jax.ShapeDtypeStruct((B, OUTW), jnp.float32),
      mesh=mesh,
      scratch_types=[
          pltpu.VMEM((CB + L,), jnp.int32),
          pltpu.VMEM((CB + L,), jnp.int32),
          pltpu.VMEM((nk + L,), jnp.int32),
          pltpu.VMEM((CB,), jnp.int32),
          pltpu.VMEM((CB,), jnp.int32),
          pltpu.VMEM((nk,), jnp.int32),
          pltpu.VMEM((CB, TD), jnp.float32),
          pltpu.VMEM((CB, TD), jnp.float32),
          pltpu.VMEM((nk, TD), jnp.float32),
          pltpu.VMEM((CB, OUTW), jnp.float32),
          pltpu.SemaphoreType.DMA,
          pltpu.SemaphoreType.DMA,
      ] + [pltpu.SemaphoreType.DMA] * n_idx_ch,

      compiler_params=pltpu.CompilerParams(use_tc_tiling_on_sc=False),
  )


def kernel(center, pos, neg, center_emb, context_emb):
  B = center.shape[0]
  K = neg.shape[1]
  V, D = center_emb.shape
  OUTW = 32
  center = center.astype(jnp.int32)
  pos = pos.astype(jnp.int32)
  negf = neg.astype(jnp.int32).reshape(B * K)
  relayout = _make_relayout(V, D)
  cemb_p, xemb_p = relayout(center_emb.T, context_emb.T)
  H = cemb_p.shape[0]
  TD = cemb_p.shape[1]
  cemb_f, xemb_f = lax.optimization_barrier(
      (cemb_p.reshape(H * TD), xemb_p.reshape(H * TD)))
  cemb_p = cemb_f.reshape(H, TD)
  xemb_p = xemb_f.reshape(H, TD)
  k = _make_sc_kernel(B, K, D, OUTW, H)
  out = k(center, pos, negf, cemb_p, xemb_p)
  return out[:, K], out[:, :K]


# speedup vs baseline: 11.4067x; 1.1049x over previous
import functools

import jax
import jax.numpy as jnp
from jax import lax
from jax.experimental import pallas as pl
from jax.experimental.pallas import tpu as pltpu
from jax.experimental.pallas import tpu_sc as plsc

NC = 2
NS = 16
NW = NC * NS
L = 16

CB = 32
IDX_CH = 128
BW = 8192

_DNUMS = lax.GatherDimensionNumbers(
    offset_dims=(), collapsed_slice_dims=(0,), start_index_map=(0,))


def _gat(x, idx):
  return lax.gather(x, idx, _DNUMS, (1,),
                    mode=lax.GatherScatterMode.PROMISE_IN_BOUNDS)


def _make_relayout(V, D):
  TD = 2 * D
  n_blocks = (V + BW - 1) // BW
  H = n_blocks * (BW // 2)

  def tbody(a_ref, b_ref, pa_ref, pb_ref):
    r = lax.broadcasted_iota(jnp.int32, (TD, TD), 0)
    c = lax.broadcasted_iota(jnp.int32, (TD, TD), 1)
    ident = jnp.where(r == c, 1.0, 0.0).astype(jnp.float32)
    dn = (((1,), (1,)), ((), ()))

    def dott(x):
      return lax.dot_general(ident, x, dn, preferred_element_type=jnp.float32)

    ya = jnp.swapaxes(a_ref[...], 0, 1)
    xb = b_ref[...]
    for j in range(BW // (2 * TD)):
      pa_ref[j * TD:(j + 1) * TD, :] = jnp.concatenate(
          [ya[j * 2 * TD:j * 2 * TD + TD], ya[j * 2 * TD + TD:(j + 1) * 2 * TD]],
          axis=1)
      yt = dott(xb[:, j * 2 * TD:j * 2 * TD + TD])
      yb2 = dott(xb[:, j * 2 * TD + TD:(j + 1) * 2 * TD])
      pb_ref[j * TD:(j + 1) * TD, :] = jnp.concatenate([yt, yb2], axis=1)

  return pl.pallas_call(
      tbody,
      grid=(n_blocks,),
      in_specs=[
          pl.BlockSpec((D, BW), lambda g: (0, g)),
          pl.BlockSpec((D, BW), lambda g: (0, g)),
      ],
      out_specs=[
          pl.BlockSpec((BW // 2, TD), lambda g: (g, 0)),
          pl.BlockSpec((BW // 2, TD), lambda g: (g, 0)),
      ],
      out_shape=[
          jax.ShapeDtypeStruct((H, TD), jnp.float32),
          jax.ShapeDtypeStruct((H, TD), jnp.float32),
      ],
      compiler_params=pltpu.CompilerParams(needs_layout_passes=False),
  )


def _make_sc_kernel(B, K, D, OUTW, H):
  b_per_w = B // NW
  n_chunks = b_per_w // CB
  nk = CB * K
  n_idx_ch = nk // IDX_CH
  nd = D // L

  mesh = plsc.VectorSubcoreMesh(core_axis_name="c", subcore_axis_name="s")

  seg = [0]
  for t in range(n_idx_ch):
    seg.append(min(CB, (IDX_CH * (t + 1)) // K))

  def body(center_hbm, pos_hbm, negf_hbm, cemb_hbm, xemb_hbm, out_hbm,
           cidx, pidx, nidx, crow, prow, nrow, vrows, prows, nrows, outbuf,
           sem_v, sem_p, *sem_n):
    wid = lax.axis_index("s") * NC + lax.axis_index("c")
    base = wid * b_per_w
    lane = lax.iota(jnp.int32, L)
    zero_idx = jnp.zeros((L, 1), jnp.int32)
    perms = [(lane ^ s).reshape(L, 1) for s in (8, 4, 2, 1)]
    kidx = [jnp.full((L, 1), kk, jnp.int32) for kk in range(L)]

    def lanesum(x):
      for idx in perms:
        x = x + _gat(x, idx)
      return x

    def pairrow(x):
      return ((x >> 8) << 8) | ((x & 127) << 1) | ((x >> 7) & 1)

    def chunk_body(c, carry):
      row0 = base + c * CB
      pltpu.sync_copy(center_hbm.at[pl.ds(row0, CB)], cidx.at[pl.ds(0, CB)])
      pltpu.sync_copy(pos_hbm.at[pl.ds(row0, CB)], pidx.at[pl.ds(0, CB)])
      pltpu.sync_copy(negf_hbm.at[pl.ds(row0 * K, nk)], nidx.at[pl.ds(0, nk)])
      for j in range(CB // L):
        crow[pl.ds(j * L, L)] = pairrow(cidx[pl.ds(j * L, L)])
        prow[pl.ds(j * L, L)] = pairrow(pidx[pl.ds(j * L, L)])
      for j in range(nk // L):
        nrow[pl.ds(j * L, L)] = pairrow(nidx[pl.ds(j * L, L)])
      cp_v = pltpu.async_copy(cemb_hbm.at[crow], vrows, sem_v)
      cp_p = pltpu.async_copy(xemb_hbm.at[prow], prows, sem_p)
      cp_n = []
      for j in range(n_idx_ch):
        cp_n.append(pltpu.async_copy(
            xemb_hbm.at[nrow.at[pl.ds(j * IDX_CH, IDX_CH)]],
            nrows.at[pl.ds(j * IDX_CH, IDX_CH)], sem_n[j]))
      cp_v.wait()
      cp_p.wait()

      def rowvecs(ref, r):
        return [ref[r, pl.ds(j * L, L)] for j in range(nd)]

      def elem_body(b, carry2):
        v = rowvecs(vrows, b)
        acc1 = jnp.zeros((L,), jnp.float32)
        acc2 = jnp.zeros((L,), jnp.float32)
        nbase = b * K
        for k in range(K):
          kk = k if k < L else k - L
          u = rowvecs(nrows, nbase + k)
          p = v[0] * u[0]
          for j in range(1, nd):
            p = p + v[j] * u[j]
          s = lanesum(p)
          if k < L:
            acc1 = jnp.where(lane == kk, s, acc1)
          else:
            acc2 = jnp.where(lane == kk, s, acc2)
        u = rowvecs(prows, b)
        p = v[0] * u[0]
        for j in range(1, nd):
          p = p + v[j] * u[j]
        s = lanesum(p)
        acc2 = jnp.where(lane == (K - L), s, acc2)
        outbuf[b, pl.ds(0, L)] = acc1
        outbuf[b, pl.ds(L, L)] = acc2
        return carry2

      for t in range(n_idx_ch):
        cp_n[t].wait()
        if seg[t + 1] > seg[t]:
          lax.fori_loop(seg[t], seg[t + 1], elem_body, 0)
      pltpu.sync_copy(outbuf, out_hbm.at[pl.ds(row0, CB)])
      return carry

    lax.fori_loop(0, n_chunks, chunk_body, 0)

  return pl.kernel(
      body,
      out_type=jax.ShapeDtypeStruct((B, OUTW), jnp.float32),
      mesh=mesh,
      scratch_types=[
          pltpu.VMEM((CB + L,), jnp.int32),
          pltpu.VMEM((CB + L,), jnp.int32),
          pltpu.VMEM((nk + L,), jnp.int32),
          pltpu.VMEM((CB,), jnp.int32),
          pltpu.VMEM((CB,), jnp.int32),
          pltpu.VMEM((nk,), jnp.int32),
          pltpu.VMEM((CB, D), jnp.float32),
          pltpu.VMEM((CB, D), jnp.float32),
          pltpu.VMEM((nk, D), jnp.float32),
          pltpu.VMEM((CB, OUTW), jnp.float32),
          pltpu.SemaphoreType.DMA,
          pltpu.SemaphoreType.DMA,
      ] + [pltpu.SemaphoreType.DMA] * n_idx_ch,

      compiler_params=pltpu.CompilerParams(use_tc_tiling_on_sc=False),
  )


def kernel(center, pos, neg, center_emb, context_emb):
  B = center.shape[0]
  K = neg.shape[1]
  V, D = center_emb.shape
  OUTW = 32
  center = center.astype(jnp.int32)
  pos = pos.astype(jnp.int32)
  negf = neg.astype(jnp.int32).reshape(B * K)
  relayout = _make_relayout(V, D)
  cemb_p, xemb_p = relayout(center_emb.T, context_emb.T)
  H = cemb_p.shape[0]
  TD = cemb_p.shape[1]
  cemb_f, xemb_f = lax.optimization_barrier(
      (cemb_p.reshape(H * TD), xemb_p.reshape(H * TD)))
  cemb_p = cemb_f.reshape(2 * H, D)
  xemb_p = xemb_f.reshape(2 * H, D)
  k = _make_sc_kernel(B, K, D, OUTW, H)
  out = k(center, pos, negf, cemb_p, xemb_p)
  return out[:, K], out[:, :K]


# speedup vs baseline: 11.9021x; 1.0434x over previous
import functools

import jax
import jax.numpy as jnp
from jax import lax
from jax.experimental import pallas as pl
from jax.experimental.pallas import tpu as pltpu
from jax.experimental.pallas import tpu_sc as plsc

NC = 2
NS = 16
NW = NC * NS
L = 16

CB = 64
IDX_CH = 128
BW = 8192

_DNUMS = lax.GatherDimensionNumbers(
    offset_dims=(), collapsed_slice_dims=(0,), start_index_map=(0,))


def _gat(x, idx):
  return lax.gather(x, idx, _DNUMS, (1,),
                    mode=lax.GatherScatterMode.PROMISE_IN_BOUNDS)


def _make_relayout(V, D):
  TD = 2 * D
  n_blocks = (V + BW - 1) // BW
  H = n_blocks * (BW // 2)

  def tbody(a_ref, b_ref, pa_ref, pb_ref):
    r = lax.broadcasted_iota(jnp.int32, (TD, TD), 0)
    c = lax.broadcasted_iota(jnp.int32, (TD, TD), 1)
    ident = jnp.where(r == c, 1.0, 0.0).astype(jnp.float32)
    dn = (((1,), (1,)), ((), ()))

    def dott(x):
      return lax.dot_general(ident, x, dn, preferred_element_type=jnp.float32)

    ya = jnp.swapaxes(a_ref[...], 0, 1)
    xb = b_ref[...]
    for j in range(BW // (2 * TD)):
      pa_ref[j * TD:(j + 1) * TD, :] = jnp.concatenate(
          [ya[j * 2 * TD:j * 2 * TD + TD], ya[j * 2 * TD + TD:(j + 1) * 2 * TD]],
          axis=1)
      yt = dott(xb[:, j * 2 * TD:j * 2 * TD + TD])
      yb2 = dott(xb[:, j * 2 * TD + TD:(j + 1) * 2 * TD])
      pb_ref[j * TD:(j + 1) * TD, :] = jnp.concatenate([yt, yb2], axis=1)

  return pl.pallas_call(
      tbody,
      grid=(n_blocks,),
      in_specs=[
          pl.BlockSpec((D, BW), lambda g: (0, g)),
          pl.BlockSpec((D, BW), lambda g: (0, g)),
      ],
      out_specs=[
          pl.BlockSpec((BW // 2, TD), lambda g: (g, 0)),
          pl.BlockSpec((BW // 2, TD), lambda g: (g, 0)),
      ],
      out_shape=[
          jax.ShapeDtypeStruct((H, TD), jnp.float32),
          jax.ShapeDtypeStruct((H, TD), jnp.float32),
      ],
      compiler_params=pltpu.CompilerParams(needs_layout_passes=False),
  )


def _make_sc_kernel(B, K, D, OUTW, H):
  b_per_w = B // NW
  n_chunks = b_per_w // CB
  nk = CB * K
  n_idx_ch = nk // IDX_CH
  nd = D // L

  mesh = plsc.VectorSubcoreMesh(core_axis_name="c", subcore_axis_name="s")

  seg = [0]
  for t in range(n_idx_ch):
    seg.append(min(CB, (IDX_CH * (t + 1)) // K))

  def body(center_hbm, pos_hbm, negf_hbm, cemb_hbm, xemb_hbm, out_hbm,
           cidx, pidx, nidx, crow, prow, nrow, vrows, prows, nrows, outbuf,
           sem_v, sem_p, *sem_n):
    wid = lax.axis_index("s") * NC + lax.axis_index("c")
    base = wid * b_per_w
    lane = lax.iota(jnp.int32, L)
    zero_idx = jnp.zeros((L, 1), jnp.int32)
    perms = [(lane ^ s).reshape(L, 1) for s in (8, 4, 2, 1)]
    kidx = [jnp.full((L, 1), kk, jnp.int32) for kk in range(L)]

    def lanesum(x):
      for idx in perms:
        x = x + _gat(x, idx)
      return x

    def pairrow(x):
      return ((x >> 8) << 8) | ((x & 127) << 1) | ((x >> 7) & 1)

    def chunk_body(c, carry):
      row0 = base + c * CB
      pltpu.sync_copy(center_hbm.at[pl.ds(row0, CB)], cidx.at[pl.ds(0, CB)])
      pltpu.sync_copy(pos_hbm.at[pl.ds(row0, CB)], pidx.at[pl.ds(0, CB)])
      pltpu.sync_copy(negf_hbm.at[pl.ds(row0 * K, nk)], nidx.at[pl.ds(0, nk)])
      for j in range(CB // L):
        crow[pl.ds(j * L, L)] = pairrow(cidx[pl.ds(j * L, L)])
        prow[pl.ds(j * L, L)] = pairrow(pidx[pl.ds(j * L, L)])
      for j in range(nk // L):
        nrow[pl.ds(j * L, L)] = pairrow(nidx[pl.ds(j * L, L)])
      cp_v = pltpu.async_copy(cemb_hbm.at[crow], vrows, sem_v)
      cp_p = pltpu.async_copy(xemb_hbm.at[prow], prows, sem_p)
      cp_n = []
      for j in range(n_idx_ch):
        cp_n.append(pltpu.async_copy(
            xemb_hbm.at[nrow.at[pl.ds(j * IDX_CH, IDX_CH)]],
            nrows.at[pl.ds(j * IDX_CH, IDX_CH)], sem_n[j]))
      cp_v.wait()
      cp_p.wait()

      def rowvecs(ref, r):
        return [ref[r, pl.ds(j * L, L)] for j in range(nd)]

      def elem_body(b, carry2):
        v = rowvecs(vrows, b)
        acc1 = jnp.zeros((L,), jnp.float32)
        acc2 = jnp.zeros((L,), jnp.float32)
        nbase = b * K
        for k in range(K):
          kk = k if k < L else k - L
          u = rowvecs(nrows, nbase + k)
          p = v[0] * u[0]
          for j in range(1, nd):
            p = p + v[j] * u[j]
          s = lanesum(p)
          if k < L:
            acc1 = jnp.where(lane == kk, s, acc1)
          else:
            acc2 = jnp.where(lane == kk, s, acc2)
        u = rowvecs(prows, b)
        p = v[0] * u[0]
        for j in range(1, nd):
          p = p + v[j] * u[j]
        s = lanesum(p)
        acc2 = jnp.where(lane == (K - L), s, acc2)
        outbuf[b, pl.ds(0, L)] = acc1
        outbuf[b, pl.ds(L, L)] = acc2
        return carry2

      for t in range(n_idx_ch):
        cp_n[t].wait()
        if seg[t + 1] > seg[t]:
          lax.fori_loop(seg[t], seg[t + 1], elem_body, 0)
      pltpu.sync_copy(outbuf, out_hbm.at[pl.ds(row0, CB)])
      return carry

    lax.fori_loop(0, n_chunks, chunk_body, 0)

  return pl.kernel(
      body,
      out_type=jax.ShapeDtypeStruct((B, OUTW), jnp.float32),
      mesh=mesh,
      scratch_types=[
          pltpu.VMEM((CB + L,), jnp.int32),
          pltpu.VMEM((CB + L,), jnp.int32),
          pltpu.VMEM((nk + L,), jnp.int32),
          pltpu.VMEM((CB,), jnp.int32),
          pltpu.VMEM((CB,), jnp.int32),
          pltpu.VMEM((nk,), jnp.int32),
          pltpu.VMEM((CB, D), jnp.float32),
          pltpu.VMEM((CB, D), jnp.float32),
          pltpu.VMEM((nk, D), jnp.float32),
          pltpu.VMEM((CB, OUTW), jnp.float32),
          pltpu.SemaphoreType.DMA,
          pltpu.SemaphoreType.DMA,
      ] + [pltpu.SemaphoreType.DMA] * n_idx_ch,

      compiler_params=pltpu.CompilerParams(use_tc_tiling_on_sc=False),
  )


def kernel(center, pos, neg, center_emb, context_emb):
  B = center.shape[0]
  K = neg.shape[1]
  V, D = center_emb.shape
  OUTW = 32
  center = center.astype(jnp.int32)
  pos = pos.astype(jnp.int32)
  negf = neg.astype(jnp.int32).reshape(B * K)
  relayout = _make_relayout(V, D)
  cemb_p, xemb_p = relayout(center_emb.T, context_emb.T)
  H = cemb_p.shape[0]
  TD = cemb_p.shape[1]
  cemb_f, xemb_f = lax.optimization_barrier(
      (cemb_p.reshape(H * TD), xemb_p.reshape(H * TD)))
  cemb_p = cemb_f.reshape(2 * H, D)
  xemb_p = xemb_f.reshape(2 * H, D)
  k = _make_sc_kernel(B, K, D, OUTW, H)
  out = k(center, pos, negf, cemb_p, xemb_p)
  return out[:, K], out[:, :K]


# speedup vs baseline: 12.7532x; 1.0715x over previous
import functools

import jax
import jax.numpy as jnp
from jax import lax
from jax.experimental import pallas as pl
from jax.experimental.pallas import tpu as pltpu
from jax.experimental.pallas import tpu_sc as plsc

NC = 2
NS = 16
NW = NC * NS
L = 16

CB = 64
IDX_CH = 128
BW = 16384

_DNUMS = lax.GatherDimensionNumbers(
    offset_dims=(), collapsed_slice_dims=(0,), start_index_map=(0,))


def _gat(x, idx):
  return lax.gather(x, idx, _DNUMS, (1,),
                    mode=lax.GatherScatterMode.PROMISE_IN_BOUNDS)


def _make_relayout(V, D):
  TD = 2 * D
  n_blocks = (V + BW - 1) // BW
  H = n_blocks * (BW // 2)

  def tbody(a_ref, b_ref, pa_ref, pb_ref):
    r = lax.broadcasted_iota(jnp.int32, (TD, TD), 0)
    c = lax.broadcasted_iota(jnp.int32, (TD, TD), 1)
    ident = jnp.where(r == c, 1.0, 0.0).astype(jnp.float32)
    dn = (((1,), (1,)), ((), ()))

    def dott(x):
      return lax.dot_general(ident, x, dn, preferred_element_type=jnp.float32)

    ya = jnp.swapaxes(a_ref[...], 0, 1)
    xb = b_ref[...]
    for j in range(BW // (2 * TD)):
      pa_ref[j * TD:(j + 1) * TD, :] = jnp.concatenate(
          [ya[j * 2 * TD:j * 2 * TD + TD], ya[j * 2 * TD + TD:(j + 1) * 2 * TD]],
          axis=1)
      yt = dott(xb[:, j * 2 * TD:j * 2 * TD + TD])
      yb2 = dott(xb[:, j * 2 * TD + TD:(j + 1) * 2 * TD])
      pb_ref[j * TD:(j + 1) * TD, :] = jnp.concatenate([yt, yb2], axis=1)

  return pl.pallas_call(
      tbody,
      grid=(n_blocks,),
      in_specs=[
          pl.BlockSpec((D, BW), lambda g: (0, g)),
          pl.BlockSpec((D, BW), lambda g: (0, g)),
      ],
      out_specs=[
          pl.BlockSpec((BW // 2, TD), lambda g: (g, 0)),
          pl.BlockSpec((BW // 2, TD), lambda g: (g, 0)),
      ],
      out_shape=[
          jax.ShapeDtypeStruct((H, TD), jnp.float32),
          jax.ShapeDtypeStruct((H, TD), jnp.float32),
      ],
      compiler_params=pltpu.CompilerParams(needs_layout_passes=False),
  )


def _make_sc_kernel(B, K, D, OUTW, H):
  b_per_w = B // NW
  n_chunks = b_per_w // CB
  nk = CB * K
  n_idx_ch = nk // IDX_CH
  nd = D // L

  mesh = plsc.VectorSubcoreMesh(core_axis_name="c", subcore_axis_name="s")

  seg = [0]
  for t in range(n_idx_ch):
    seg.append(min(CB, (IDX_CH * (t + 1)) // K))

  def body(center_hbm, pos_hbm, negf_hbm, cemb_hbm, xemb_hbm, out_hbm,
           cidx, pidx, nidx, crow, prow, nrow, vrows, prows, nrows, outbuf,
           sem_v, sem_p, *sem_n):
    wid = lax.axis_index("s") * NC + lax.axis_index("c")
    base = wid * b_per_w
    lane = lax.iota(jnp.int32, L)
    zero_idx = jnp.zeros((L, 1), jnp.int32)
    perms = [(lane ^ s).reshape(L, 1) for s in (8, 4, 2, 1)]
    kidx = [jnp.full((L, 1), kk, jnp.int32) for kk in range(L)]

    def lanesum(x):
      for idx in perms:
        x = x + _gat(x, idx)
      return x

    def pairrow(x):
      return ((x >> 8) << 8) | ((x & 127) << 1) | ((x >> 7) & 1)

    def chunk_body(c, carry):
      row0 = base + c * CB
      pltpu.sync_copy(center_hbm.at[pl.ds(row0, CB)], cidx.at[pl.ds(0, CB)])
      pltpu.sync_copy(pos_hbm.at[pl.ds(row0, CB)], pidx.at[pl.ds(0, CB)])
      pltpu.sync_copy(negf_hbm.at[pl.ds(row0 * K, nk)], nidx.at[pl.ds(0, nk)])
      for j in range(CB // L):
        crow[pl.ds(j * L, L)] = pairrow(cidx[pl.ds(j * L, L)])
        prow[pl.ds(j * L, L)] = pairrow(pidx[pl.ds(j * L, L)])
      for j in range(nk // L):
        nrow[pl.ds(j * L, L)] = pairrow(nidx[pl.ds(j * L, L)])
      cp_v = pltpu.async_copy(cemb_hbm.at[crow], vrows, sem_v)
      cp_p = pltpu.async_copy(xemb_hbm.at[prow], prows, sem_p)
      cp_n = []
      for j in range(n_idx_ch):
        cp_n.append(pltpu.async_copy(
            xemb_hbm.at[nrow.at[pl.ds(j * IDX_CH, IDX_CH)]],
            nrows.at[pl.ds(j * IDX_CH, IDX_CH)], sem_n[j]))
      cp_v.wait()
      cp_p.wait()

      def rowvecs(ref, r):
        return [ref[r, pl.ds(j * L, L)] for j in range(nd)]

      def elem_body(b, carry2):
        v = rowvecs(vrows, b)
        acc1 = jnp.zeros((L,), jnp.float32)
        acc2 = jnp.zeros((L,), jnp.float32)
        nbase = b * K
        for k in range(K):
          kk = k if k < L else k - L
          u = rowvecs(nrows, nbase + k)
          p = v[0] * u[0]
          for j in range(1, nd):
            p = p + v[j] * u[j]
          s = lanesum(p)
          if k < L:
            acc1 = jnp.where(lane == kk, s, acc1)
          else:
            acc2 = jnp.where(lane == kk, s, acc2)
        u = rowvecs(prows, b)
        p = v[0] * u[0]
        for j in range(1, nd):
          p = p + v[j] * u[j]
        s = lanesum(p)
        acc2 = jnp.where(lane == (K - L), s, acc2)
        outbuf[b, pl.ds(0, L)] = acc1
        outbuf[b, pl.ds(L, L)] = acc2
        return carry2

      for t in range(n_idx_ch):
        cp_n[t].wait()
        if seg[t + 1] > seg[t]:
          lax.fori_loop(seg[t], seg[t + 1], elem_body, 0)
      pltpu.sync_copy(outbuf, out_hbm.at[pl.ds(row0, CB)])
      return carry

    lax.fori_loop(0, n_chunks, chunk_body, 0)

  return pl.kernel(
      body,
      out_type=jax.ShapeDtypeStruct((B, OUTW), jnp.float32),
      mesh=mesh,
      scratch_types=[
          pltpu.VMEM((CB + L,), jnp.int32),
          pltpu.VMEM((CB + L,), jnp.int32),
          pltpu.VMEM((nk + L,), jnp.int32),
          pltpu.VMEM((CB,), jnp.int32),
          pltpu.VMEM((CB,), jnp.int32),
          pltpu.VMEM((nk,), jnp.int32),
          pltpu.VMEM((CB, D), jnp.float32),
          pltpu.VMEM((CB, D), jnp.float32),
          pltpu.VMEM((nk, D), jnp.float32),
          pltpu.VMEM((CB, OUTW), jnp.float32),
          pltpu.SemaphoreType.DMA,
          pltpu.SemaphoreType.DMA,
      ] + [pltpu.SemaphoreType.DMA] * n_idx_ch,

      compiler_params=pltpu.CompilerParams(use_tc_tiling_on_sc=False),
  )


def kernel(center, pos, neg, center_emb, context_emb):
  B = center.shape[0]
  K = neg.shape[1]
  V, D = center_emb.shape
  OUTW = 32
  center = center.astype(jnp.int32)
  pos = pos.astype(jnp.int32)
  negf = neg.astype(jnp.int32).reshape(B * K)
  relayout = _make_relayout(V, D)
  cemb_p, xemb_p = relayout(center_emb.T, context_emb.T)
  H = cemb_p.shape[0]
  TD = cemb_p.shape[1]
  cemb_f, xemb_f = lax.optimization_barrier(
      (cemb_p.reshape(H * TD), xemb_p.reshape(H * TD)))
  cemb_p = cemb_f.reshape(2 * H, D)
  xemb_p = xemb_f.reshape(2 * H, D)
  k = _make_sc_kernel(B, K, D, OUTW, H)
  out = k(center, pos, negf, cemb_p, xemb_p)
  return out[:, K], out[:, :K]
